# Initial kernel scaffold; baseline (speedup 1.0000x reference)
import jax
import jax.numpy as jnp
from jax.experimental import pallas as pl


def kernel(bond_types, distances, bond_type_bias, distance_bias):
    raise NotImplementedError("write your pallas kernel here")


# speedup vs baseline: 13.1538x; 13.1538x over previous
import functools

import jax
import jax.numpy as jnp
from jax import lax
from jax.experimental import pallas as pl
from jax.experimental.pallas import tpu as pltpu, tpu_sc as plsc

N_HEADS = 8
N_BOND = 5
N_DIST = 7
NCOMB = N_BOND * N_DIST

_info = plsc.get_sparse_core_info()
NC, NS, L = _info.num_cores, _info.num_subcores, _info.num_lanes
NW = NC * NS

EDGES = 512 * 128 * 128
E_PER_W = EDGES // NW
CHUNK = 4096
NCH = E_PER_W // CHUNK
OUT_CHUNK = CHUNK * N_HEADS


def _sc_body(bond_h, dist_h, btab_h, dtab_h, out_h,
             bond_v, dist_v, btab_v, dtab_v, tcomb_v, c8_v, out_v):
    wid = lax.axis_index("s") * NC + lax.axis_index("c")

    iota = lax.iota(jnp.int32, L)
    h_off = iota & 7
    e_off = iota >> 3

    pltpu.sync_copy(btab_h, btab_v.at[pl.ds(0, N_BOND * N_HEADS)])
    pltpu.sync_copy(dtab_h, dtab_v.at[pl.ds(0, N_DIST * N_HEADS)])
    for k in range(18):
        t = iota + 16 * k
        c = t >> 3
        h = t & 7
        b = (c * 9363) >> 16
        d = c - b * 7
        vb = plsc.load_gather(btab_v, [b * 8 + h])
        vd = plsc.load_gather(dtab_v, [d * 8 + h])
        tcomb_v[pl.ds(16 * k, L)] = vb + vd

    def chunk_body(g, carry):
        base = wid * E_PER_W + g * CHUNK
        pltpu.sync_copy(bond_h.at[pl.ds(base, CHUNK)], bond_v)
        pltpu.sync_copy(dist_h.at[pl.ds(base, CHUNK)], dist_v)

        def prep(j, c2):
            b = bond_v[pl.ds(16 * j, L)]
            d = dist_v[pl.ds(16 * j, L)]
            c8_v[pl.ds(16 * j, L)] = (b * 7 + d) * 8
            return c2

        lax.fori_loop(0, CHUNK // L, prep, 0, unroll=4)

        def main(i, c2):
            e_vec = 2 * i + e_off
            a = plsc.load_gather(c8_v, [e_vec])
            v = plsc.load_gather(tcomb_v, [a + h_off])
            out_v[pl.ds(16 * i, L)] = v
            return c2

        lax.fori_loop(0, OUT_CHUNK // L, main, 0, unroll=4)
        pltpu.sync_copy(out_v, out_h.at[pl.ds(base * N_HEADS, OUT_CHUNK)])
        return carry

    lax.fori_loop(0, NCH, chunk_body, 0)


@jax.jit
def _sc_call(bond_f, dist_f, btab, dtab):
    mesh = plsc.VectorSubcoreMesh(core_axis_name="c", subcore_axis_name="s")
    return pl.kernel(
        _sc_body,
        out_type=jax.ShapeDtypeStruct((EDGES * N_HEADS,), jnp.float32),
        mesh=mesh,
        compiler_params=pltpu.CompilerParams(needs_layout_passes=False),
        scratch_types=[
            pltpu.VMEM((CHUNK,), jnp.int32),
            pltpu.VMEM((CHUNK,), jnp.int32),
            pltpu.VMEM((64,), jnp.float32),
            pltpu.VMEM((64,), jnp.float32),
            pltpu.VMEM((288,), jnp.float32),
            pltpu.VMEM((CHUNK,), jnp.int32),
            pltpu.VMEM((OUT_CHUNK,), jnp.float32),
        ],
    )(bond_f, dist_f, btab, dtab)


def kernel(bond_types, distances, bond_type_bias, distance_bias):
    shape = bond_types.shape
    out = _sc_call(
        bond_types.reshape(-1).astype(jnp.int32),
        distances.reshape(-1).astype(jnp.int32),
        bond_type_bias.reshape(-1).astype(jnp.float32),
        distance_bias.reshape(-1).astype(jnp.float32),
    )
    return out.reshape(*shape, N_HEADS)


# speedup vs baseline: 18.8438x; 1.4326x over previous
import jax
import jax.numpy as jnp
from jax import lax
from jax.experimental import pallas as pl
from jax.experimental.pallas import tpu as pltpu, tpu_sc as plsc

N_HEADS = 8
N_BOND = 5
N_DIST = 7

_info = plsc.get_sparse_core_info()
NC, NS, L = _info.num_cores, _info.num_subcores, _info.num_lanes
NW = NC * NS

EDGES = 512 * 128 * 128
E_PER_W = EDGES // NW
CHUNK = 4096
NCH = E_PER_W // CHUNK
OUT_CHUNK = CHUNK * N_HEADS


def _sc_body(bond_h, dist_h, btab_h, dtab_h, out_h,
             bond_v0, bond_v1, dist_v0, dist_v1, btab_v, dtab_v, tcomb_v,
             c8_v, out_v0, out_v1,
             sem_in0, sem_in1, sem_out0, sem_out1):
    wid = lax.axis_index("s") * NC + lax.axis_index("c")
    bond_v = (bond_v0, bond_v1)
    dist_v = (dist_v0, dist_v1)
    out_v = (out_v0, out_v1)
    sem_in = (sem_in0, sem_in1)
    sem_out = (sem_out0, sem_out1)

    iota = lax.iota(jnp.int32, L)
    h_off = iota & 7
    e_off = iota >> 3

    pltpu.sync_copy(btab_h, btab_v.at[pl.ds(0, N_BOND * N_HEADS)])
    pltpu.sync_copy(dtab_h, dtab_v.at[pl.ds(0, N_DIST * N_HEADS)])
    for k in range(18):
        t = iota + 16 * k
        c = t >> 3
        h = t & 7
        b = (c * 9363) >> 16
        d = c - b * 7
        vb = plsc.load_gather(btab_v, [b * 8 + h])
        vd = plsc.load_gather(dtab_v, [d * 8 + h])
        tcomb_v[pl.ds(16 * k, L)] = vb + vd

    def in_dma(g, p):
        base = wid * E_PER_W + g * CHUNK
        return (
            pltpu.make_async_copy(
                bond_h.at[pl.ds(base, CHUNK)], bond_v[p], sem_in[p]),
            pltpu.make_async_copy(
                dist_h.at[pl.ds(base, CHUNK)], dist_v[p], sem_in[p]),
        )

    def out_dma(g, p):
        base = wid * E_PER_W + g * CHUNK
        return pltpu.make_async_copy(
            out_v[p], out_h.at[pl.ds(base * N_HEADS, OUT_CHUNK)],
            sem_out[p])

    for p in range(2):
        for d in in_dma(p, p):
            d.start()

    def chunk(g, p):
        for d in in_dma(g, p):
            d.wait()

        bond_p, dist_p, out_p = bond_v[p], dist_v[p], out_v[p]

        @plsc.parallel_loop(0, CHUNK // L, unroll=8)
        def prep(j):
            b = bond_p[pl.ds(16 * j, L)]
            d = dist_p[pl.ds(16 * j, L)]
            c8_v[pl.ds(16 * j, L)] = (b * 7 + d) * 8

        @pl.when(g + 2 < NCH)
        def _():
            for dsc in in_dma(g + 2, p):
                dsc.start()

        @pl.when(g >= 2)
        def _():
            out_dma(g - 2, p).wait()

        @plsc.parallel_loop(0, OUT_CHUNK // L, unroll=8)
        def main(i):
            e_vec = 2 * i + e_off
            a = plsc.load_gather(c8_v, [e_vec])
            v = plsc.load_gather(tcomb_v, [a + h_off])
            out_p[pl.ds(16 * i, L)] = v

        out_dma(g, p).start()

    def pair(go, carry):
        chunk(2 * go, 0)
        chunk(2 * go + 1, 1)
        return carry

    lax.fori_loop(0, NCH // 2, pair, 0)
    out_dma(NCH - 2, 0).wait()
    out_dma(NCH - 1, 1).wait()


@jax.jit
def _sc_call(bond_f, dist_f, btab, dtab):
    mesh = plsc.VectorSubcoreMesh(core_axis_name="c", subcore_axis_name="s")
    return pl.kernel(
        _sc_body,
        out_type=jax.ShapeDtypeStruct((EDGES * N_HEADS,), jnp.float32),
        mesh=mesh,
        compiler_params=pltpu.CompilerParams(needs_layout_passes=False),
        scratch_types=[
            pltpu.VMEM((CHUNK,), jnp.int32),
            pltpu.VMEM((CHUNK,), jnp.int32),
            pltpu.VMEM((CHUNK,), jnp.int32),
            pltpu.VMEM((CHUNK,), jnp.int32),
            pltpu.VMEM((64,), jnp.float32),
            pltpu.VMEM((64,), jnp.float32),
            pltpu.VMEM((288,), jnp.float32),
            pltpu.VMEM((CHUNK,), jnp.int32),
            pltpu.VMEM((OUT_CHUNK,), jnp.float32),
            pltpu.VMEM((OUT_CHUNK,), jnp.float32),
            pltpu.SemaphoreType.DMA,
            pltpu.SemaphoreType.DMA,
            pltpu.SemaphoreType.DMA,
            pltpu.SemaphoreType.DMA,
        ],
    )(bond_f, dist_f, btab, dtab)


def kernel(bond_types, distances, bond_type_bias, distance_bias):
    shape = bond_types.shape
    out = _sc_call(
        bond_types.reshape(-1).astype(jnp.int32),
        distances.reshape(-1).astype(jnp.int32),
        bond_type_bias.reshape(-1).astype(jnp.float32),
        distance_bias.reshape(-1).astype(jnp.float32),
    )
    return out.reshape(*shape, N_HEADS)
